# baseline (speedup 1.0000x reference)
import functools

import jax
import jax.numpy as jnp
from jax import lax
from jax.experimental import pallas as pl
from jax.experimental.pallas import tpu as pltpu
from jax.experimental.pallas import tpu_sc as plsc

_NUM_CLASSES = 1000000
_NUM_SAMPLED = 8192
_S = 32
_D = 64
_NW = 32
_PER_W = _NUM_SAMPLED // _NW


def _sc_gather(W, b, sampled, lab):
    mesh = plsc.VectorSubcoreMesh(core_axis_name="c", subcore_axis_name="s")

    @functools.partial(
        pl.kernel,
        mesh=mesh,
        compiler_params=pltpu.CompilerParams(
            use_tc_tiling_on_sc=False, needs_layout_passes=False),
        out_type=[
            jax.ShapeDtypeStruct((_NUM_SAMPLED, _D), jnp.float32),
            jax.ShapeDtypeStruct((_NUM_SAMPLED,), jnp.float32),
            jax.ShapeDtypeStruct((_S, _D), jnp.float32),
            jax.ShapeDtypeStruct((_S,), jnp.float32),
        ],
        scratch_types=[
            pltpu.VMEM((_PER_W,), jnp.int32),
            pltpu.VMEM((_PER_W, _D), jnp.float32),
            pltpu.VMEM((_PER_W,), jnp.float32),
            pltpu.VMEM((_S,), jnp.int32),
            pltpu.VMEM((_S, _D), jnp.float32),
            pltpu.VMEM((_S,), jnp.float32),
            pltpu.SemaphoreType.DMA,
            pltpu.SemaphoreType.DMA,
        ],
    )
    def k(W_hbm, b_hbm, idx_hbm, lab_hbm, sw_hbm, sb_hbm, tw_hbm, tb_hbm,
          idx_v, rows_v, bv_v, lab_v, trow_v, tbv_v, sem, bsem):
        wid = lax.axis_index("s") * 2 + lax.axis_index("c")
        base = wid * _PER_W
        pltpu.sync_copy(idx_hbm.at[pl.ds(base, _PER_W)], idx_v)
        bcp = pltpu.make_async_copy(b_hbm.at[idx_v], bv_v, bsem)
        bcp.start()
        pltpu.make_async_copy(W_hbm.at[idx_v], rows_v, sem).start()
        pltpu.make_async_copy(W_hbm.at[idx_v], rows_v, sem).wait()
        bcp.wait()
        pltpu.sync_copy(rows_v, sw_hbm.at[pl.ds(base, _PER_W)])
        pltpu.sync_copy(bv_v, sb_hbm.at[pl.ds(base, _PER_W)])

        @pl.when(wid == 0)
        def _():
            pltpu.sync_copy(lab_hbm, lab_v)
            tbcp = pltpu.make_async_copy(b_hbm.at[lab_v], tbv_v, bsem)
            tbcp.start()
            pltpu.make_async_copy(W_hbm.at[lab_v], trow_v, sem).start()
            pltpu.make_async_copy(W_hbm.at[lab_v], trow_v, sem).wait()
            tbcp.wait()
            pltpu.sync_copy(trow_v, tw_hbm)
            pltpu.sync_copy(tbv_v, tb_hbm)

    return k(W, b, sampled, lab)


def _tc_loss_body(h_ref, sw_ref, sb_ref, tw_ref, tb_ref, sidx_ref, lab_ref,
                  out_ref):
    log_range = jnp.log(jnp.float32(_NUM_CLASSES + 1.0))
    h = h_ref[...]
    sw = sw_ref[...]
    dn = (((1,), (1,)), ((), ()))
    logits = lax.dot_general(h, sw, dn, preferred_element_type=jnp.float32)

    sidx = sidx_ref[...]
    c = sidx.astype(jnp.float32)
    samp_lp = jnp.log(
        jnp.log((c + 2.0) / (c + 1.0)) / log_range * _NUM_SAMPLED + 1e-12)
    logits = logits + sb_ref[...] - samp_lp

    lab = lab_ref[...]
    lc = lab.astype(jnp.float32)
    true_lp = jnp.log(
        jnp.log((lc + 2.0) / (lc + 1.0)) / log_range * _NUM_SAMPLED + 1e-12)
    t = (jnp.sum(h * tw_ref[...], axis=1, keepdims=True)
         + tb_ref[...] - true_lp)

    m = jnp.maximum(jnp.max(logits, axis=1, keepdims=True), t)
    ssum = jnp.exp(t - m) + jnp.sum(jnp.exp(logits - m), axis=1, keepdims=True)
    loss = m + jnp.log(ssum) - t
    out_ref[...] = (0.5 * jnp.mean(loss))[None, None]


def _tc_loss(h0, samp_w, samp_b, true_w, true_b, sampled, lab):
    out = pl.pallas_call(
        _tc_loss_body,
        out_shape=jax.ShapeDtypeStruct((1, 1), jnp.float32),
    )(h0, samp_w, samp_b.reshape(1, _NUM_SAMPLED), true_w,
      true_b.reshape(_S, 1), sampled.reshape(1, _NUM_SAMPLED),
      lab.reshape(_S, 1))
    return out[0, 0]


def kernel(hidden, labels, W, b):
    keys = jax.random.split(jax.random.key(42), hidden.shape[0])
    u = jax.random.uniform(keys[0], (_NUM_SAMPLED,), dtype=jnp.float32)
    s = jnp.exp(u * jnp.log(float(_NUM_CLASSES) + 1.0)) - 1.0
    sampled = jnp.clip(s.astype(jnp.int32), 0, _NUM_CLASSES - 1)
    lab = labels[0].reshape(-1).astype(jnp.int32)
    h0 = hidden[0]

    samp_w, samp_b, true_w, true_b = _sc_gather(W, b, sampled, lab)
    return _tc_loss(h0, samp_w, samp_b, true_w, true_b, sampled, lab)
